# baseline (speedup 1.0000x reference)
import functools

import jax
import jax.numpy as jnp
import numpy as np
from jax import lax
from jax.experimental import pallas as pl
from jax.experimental.pallas import tpu as pltpu
from jax.experimental.pallas import tpu_sc as plsc

N = 256
B = 1024
OUT = N * (N + 1) // 2
BANDS = 4
RB = N // BANDS
NWORKERS = 32
MPW = B // NWORKERS

_I0 = [RB * k for k in range(BANDS)]
_W = [N - RB * k for k in range(BANDS)]
_SZ = [RB * w - RB * (RB - 1) // 2 for w in _W]
_OFFS = np.cumsum([0] + _SZ).tolist()


def _build_packed_idx() -> np.ndarray:
    parts = []
    for k in range(BANDS):
        w = _W[k]
        r, c = np.meshgrid(np.arange(RB), np.arange(w), indexing="ij")
        keep = c >= r
        parts.append((r[keep].astype(np.int32) << 8) | c[keep].astype(np.int32))
    return np.concatenate(parts)


_PACKED_IDX = _build_packed_idx()


@functools.cache
def _get_triu_sc():
    mesh = plsc.VectorSubcoreMesh(core_axis_name="c", subcore_axis_name="s")

    @functools.partial(
        pl.kernel,
        out_type=jax.ShapeDtypeStruct((B, OUT), jnp.float32),
        mesh=mesh,
        scratch_types=[
            pltpu.VMEM((OUT,), jnp.int32),
            pltpu.VMEM((RB, _W[0]), jnp.float32),
            pltpu.VMEM((RB, _W[1]), jnp.float32),
            pltpu.VMEM((RB, _W[2]), jnp.float32),
            pltpu.VMEM((RB, _W[3]), jnp.float32),
            pltpu.VMEM((_SZ[0],), jnp.float32),
        ],
        compiler_params=pltpu.CompilerParams(
            use_tc_tiling_on_sc=False, needs_layout_passes=False
        ),
    )
    def _triu_sc(x_hbm, idx_hbm, out_hbm, idx_v, st0, st1, st2, st3, pk):
        staged = [st0, st1, st2, st3]
        wid = lax.axis_index("s") * 2 + lax.axis_index("c")

        pltpu.sync_copy(idx_hbm, idx_v)

        def per_matrix(j, carry):
            b = wid * MPW + j
            for k in range(BANDS):
                i0, w, sz, off = _I0[k], _W[k], _SZ[k], _OFFS[k]
                st = staged[k]
                pltpu.sync_copy(x_hbm.at[b, pl.ds(i0, RB), pl.ds(i0, w)], st)

                def chunk(t, carry2, st=st, off=off):
                    ip = idx_v[pl.ds(off + t * 16, 16)]
                    r = lax.shift_right_logical(ip, 8)
                    c = jnp.bitwise_and(ip, 255)
                    pk[pl.ds(t * 16, 16)] = plsc.load_gather(st, [r, c])
                    return carry2

                lax.fori_loop(0, sz // 16, chunk, 0)
                pltpu.sync_copy(pk.at[pl.ds(0, sz)], out_hbm.at[b, pl.ds(off, sz)])
            return carry

        lax.fori_loop(0, MPW, per_matrix, 0)

    return _triu_sc


def kernel(inputs):
    idx = jnp.asarray(_PACKED_IDX)
    return _get_triu_sc()(inputs, idx)


# speedup vs baseline: 1.7170x; 1.7170x over previous
import functools

import jax
import jax.numpy as jnp
import numpy as np
from jax import lax
from jax.experimental import pallas as pl
from jax.experimental.pallas import tpu as pltpu
from jax.experimental.pallas import tpu_sc as plsc

N = 256
B = 1024
OUT = N * (N + 1) // 2
BANDS = 4
RB = N // BANDS
NWORKERS = 32
MPW = B // NWORKERS
UNROLL = 8

_I0 = [RB * k for k in range(BANDS)]
_W = [N - RB * k for k in range(BANDS)]
_SZ = [RB * w - RB * (RB - 1) // 2 for w in _W]
_OFFS = np.cumsum([0] + _SZ).tolist()
_NCH = [-(-sz // 16 // UNROLL) * UNROLL for sz in _SZ]
_POFF = np.cumsum([0] + [16 * n for n in _NCH]).tolist()
_PKWORDS = max(16 * n for n in _NCH)


def _build_packed_idx() -> np.ndarray:
    parts = []
    for k in range(BANDS):
        w = _W[k]
        r, c = np.meshgrid(np.arange(RB), np.arange(w), indexing="ij")
        keep = c >= r
        band = (r[keep].astype(np.int32) << 8) | c[keep].astype(np.int32)
        parts.append(band)
        parts.append(np.zeros(16 * _NCH[k] - band.size, np.int32))
    return np.concatenate(parts)


_PACKED_IDX = _build_packed_idx()


@functools.cache
def _get_triu_sc():
    mesh = plsc.VectorSubcoreMesh(core_axis_name="c", subcore_axis_name="s")

    @functools.partial(
        pl.kernel,
        out_type=jax.ShapeDtypeStruct((B, OUT), jnp.float32),
        mesh=mesh,
        scratch_types=[
            pltpu.VMEM((_POFF[-1],), jnp.int32),
            pltpu.VMEM((RB, _W[0]), jnp.float32),
            pltpu.VMEM((RB, _W[1]), jnp.float32),
            pltpu.VMEM((RB, _W[2]), jnp.float32),
            pltpu.VMEM((RB, _W[3]), jnp.float32),
            pltpu.VMEM((_PKWORDS,), jnp.float32),
            pltpu.VMEM((_PKWORDS,), jnp.float32),
            pltpu.SemaphoreType.DMA,
            pltpu.SemaphoreType.DMA,
            pltpu.SemaphoreType.DMA,
            pltpu.SemaphoreType.DMA,
            pltpu.SemaphoreType.DMA,
            pltpu.SemaphoreType.DMA,
        ],
        compiler_params=pltpu.CompilerParams(
            use_tc_tiling_on_sc=False, needs_layout_passes=False
        ),
    )
    def _triu_sc(x_hbm, idx_hbm, out_hbm, idx_v, st0, st1, st2, st3,
                 pk0, pk1, si0, si1, si2, si3, so0, so1):
        st = [st0, st1, st2, st3]
        pk = [pk0, pk1]
        sem_in = [si0, si1, si2, si3]
        sem_out = [so0, so1]
        wid = lax.axis_index("s") * 2 + lax.axis_index("c")

        def in_copy(b, k):
            i0, w = _I0[k], _W[k]
            return pltpu.make_async_copy(
                x_hbm.at[b, pl.ds(i0, RB), pl.ds(i0, w)],
                st[k],
                sem_in[k],
            )

        def out_copy(b, k, slot):
            sz, off = _SZ[k], _OFFS[k]
            return pltpu.make_async_copy(
                pk[slot].at[pl.ds(0, sz)],
                out_hbm.at[b, pl.ds(off, sz)],
                sem_out[slot],
            )

        pltpu.sync_copy(idx_hbm, idx_v)
        in_copy(wid * MPW, 0).start()

        def per_matrix(j, carry):
            b = wid * MPW + j
            bn = jnp.minimum(b + 1, B - 1)
            for k in range(BANDS):
                slot = k % 2
                in_copy(b, k).wait()
                if k + 1 < BANDS:
                    in_copy(b, k + 1).start()
                else:
                    in_copy(bn, 0).start()

                prev_k = (k + 2) % BANDS
                prev_b = b if k >= 2 else b - 1

                @pl.when(jnp.logical_or(j > 0, k >= 2))
                def _wait_prev():
                    out_copy(prev_b, prev_k, slot).wait()

                poff, pkref, stref = _POFF[k], pk[slot], st[k]

                @plsc.parallel_loop(0, _NCH[k], 1, unroll=UNROLL)
                def _gather(t, poff=poff, pkref=pkref, stref=stref):
                    ip = idx_v[pl.ds(poff + t * 16, 16)]
                    r = lax.shift_right_logical(ip, 8)
                    c = jnp.bitwise_and(ip, 255)
                    pkref[pl.ds(t * 16, 16)] = plsc.load_gather(stref, [r, c])

                out_copy(b, k, slot).start()
            return carry

        lax.fori_loop(0, MPW, per_matrix, 0)

        bl = wid * MPW + MPW - 1
        out_copy(bl, 2, 0).wait()
        out_copy(bl, 3, 1).wait()
        in_copy(bl, 0).wait()

    return _triu_sc


def kernel(inputs):
    idx = jnp.asarray(_PACKED_IDX)
    return _get_triu_sc()(inputs, idx)


# speedup vs baseline: 4.6074x; 2.6834x over previous
import functools

import jax
import jax.numpy as jnp
import numpy as np
from jax import lax
from jax.experimental import pallas as pl
from jax.experimental.pallas import tpu as pltpu
from jax.experimental.pallas import tpu_sc as plsc

N = 256
B = 1024
OUT = N * (N + 1) // 2
ROUNDS = 32
PSPLIT = 25856
PB = OUT - PSPLIT

_AR, _AC = 144, 256
_BR0, _BR, _BC0, _BC = 136, 120, 128, 128

_NPA = PSPLIT // 32
_NPB = PB // 32
UNROLL = 4

_ROWOFF = np.concatenate([[0], np.cumsum(N - np.arange(N))]).astype(np.int64)


def _build_idx() -> np.ndarray:
    p = np.arange(OUT)
    i = np.searchsorted(_ROWOFF, p, side="right") - 1
    col = i + (p - _ROWOFF[i])
    r = np.where(p < PSPLIT, i, i - _BR0)
    c = np.where(p < PSPLIT, col, col - _BC0)
    u = ((r << 8) | c).astype(np.int64)
    words = []
    for lo, n in ((0, PSPLIT), (PSPLIT, PB)):
        seg = u[lo:lo + n].reshape(-1, 2, 16)
        words.append(seg[:, 0, :] | (seg[:, 1, :] << 16))
    return np.concatenate(words).reshape(-1).astype(np.int32)


_IDX = _build_idx()


@functools.cache
def _get_triu_sc():
    mesh = plsc.VectorSubcoreMesh(core_axis_name="c", subcore_axis_name="s")

    @functools.partial(
        pl.kernel,
        out_type=jax.ShapeDtypeStruct((B, OUT), jnp.float32),
        mesh=mesh,
        scratch_types=[
            pltpu.VMEM((_IDX.size,), jnp.int32),
            pltpu.VMEM((_AR, _AC), jnp.float32),
            pltpu.VMEM((_BR, _BC), jnp.float32),
            pltpu.VMEM((PSPLIT,), jnp.float32),
            pltpu.VMEM_SHARED((2, 8, OUT), jnp.float32),
            pltpu.SemaphoreType.DMA,
            pltpu.SemaphoreType.DMA,
            pltpu.SemaphoreType.DMA,
        ],
        compiler_params=pltpu.CompilerParams(
            use_tc_tiling_on_sc=True, needs_layout_passes=False
        ),
    )
    def _triu_sc(x_hbm, idx_hbm, out_hbm, idx_v, sta, stb, pk, blk,
                 sem_a, sem_b, sem_out):
        c = lax.axis_index("c")
        s = lax.axis_index("s")
        h = s // 8
        m = s % 8
        gbase = (c * 2 + h) * ROUNDS

        def in_a(r):
            return pltpu.make_async_copy(
                x_hbm.at[(gbase + r) * 8 + m, pl.ds(0, _AR), pl.ds(0, _AC)],
                sta, sem_a)

        def in_b(r):
            return pltpu.make_async_copy(
                x_hbm.at[(gbase + r) * 8 + m, pl.ds(_BR0, _BR),
                         pl.ds(_BC0, _BC)],
                stb, sem_b)

        def out_copy(r):
            return pltpu.make_async_copy(
                blk.at[h],
                out_hbm.at[pl.ds((gbase + r) * 8, 8), pl.ds(0, OUT)],
                sem_out)

        pltpu.sync_copy(idx_hbm, idx_v)
        in_a(0).start()
        in_b(0).start()

        def gather(woff, npairs, stref):
            @plsc.parallel_loop(0, npairs, 1, unroll=UNROLL)
            def _g(t, woff=woff, stref=stref):
                w = idx_v[pl.ds(woff + t * 16, 16)]
                for half in range(2):
                    v = (jnp.bitwise_and(w, 0xFFFF) if half == 0
                         else lax.shift_right_logical(w, 16))
                    r = lax.shift_right_logical(v, 8)
                    cc = jnp.bitwise_and(v, 255)
                    pk[pl.ds(t * 32 + half * 16, 16)] = (
                        plsc.load_gather(stref, [r, cc]))

        def per_round(r, carry):
            rn = jnp.minimum(r + 1, ROUNDS - 1)
            in_a(r).wait()
            gather(0, _NPA, sta)
            in_a(rn).start()

            @pl.when(jnp.logical_and(m == 0, r >= 1))
            def _wait_prev():
                out_copy(r - 1).wait()

            plsc.subcore_barrier()
            pltpu.sync_copy(pk.at[pl.ds(0, PSPLIT)],
                            blk.at[h, m, pl.ds(0, PSPLIT)])
            in_b(r).wait()
            gather(_NPA * 16, _NPB, stb)
            in_b(rn).start()
            pltpu.sync_copy(pk.at[pl.ds(0, PB)],
                            blk.at[h, m, pl.ds(PSPLIT, PB)])
            plsc.subcore_barrier()

            @pl.when(m == 0)
            def _emit():
                out_copy(r).start()

            return carry

        lax.fori_loop(0, ROUNDS, per_round, 0)

        @pl.when(m == 0)
        def _drain():
            out_copy(ROUNDS - 1).wait()

        in_a(ROUNDS - 1).wait()
        in_b(ROUNDS - 1).wait()

    return _triu_sc


def kernel(inputs):
    idx = jnp.asarray(_IDX)
    return _get_triu_sc()(inputs, idx)
